# speedup vs baseline: 5.1805x; 5.1805x over previous
import functools

import jax
import jax.numpy as jnp
from jax import lax
from jax.experimental import pallas as pl
from jax.experimental.pallas import tpu as pltpu
from jax.experimental.pallas import tpu_sc as plsc

D = 128
NW = 32
CH = 128


def _gather_sc(idx, table, n):
    per_w = n // NW
    n_ch = per_w // CH

    mesh = plsc.VectorSubcoreMesh(core_axis_name="c", subcore_axis_name="s")

    @functools.partial(
        pl.kernel,
        mesh=mesh,
        out_type=jax.ShapeDtypeStruct((n, D), jnp.float32),
        scratch_types=[
            pltpu.VMEM((CH,), jnp.int32),
            pltpu.VMEM((CH, D), jnp.float32),
            pltpu.SemaphoreType.DMA,
        ],
    )
    def k(idx_hbm, table_hbm, out_hbm, idx_v, rows_v, sem):
        wid = lax.axis_index("s") * 2 + lax.axis_index("c")
        base = wid * per_w

        def body(i, carry):
            off = base + i * CH
            pltpu.sync_copy(idx_hbm.at[pl.ds(off, CH)], idx_v)
            pltpu.async_copy(table_hbm.at[idx_v], rows_v, sem).wait()
            pltpu.sync_copy(rows_v, out_hbm.at[pl.ds(off, CH)])
            return carry

        lax.fori_loop(0, n_ch, body, 0)

    return k(idx, table)


def kernel(x, infile):
    b, t = x.shape
    n = b * t
    idx = x.reshape(n).astype(jnp.int32)
    out = _gather_sc(idx, infile, n)
    return out.reshape(b, t, D)


# speedup vs baseline: 9.0710x; 1.7510x over previous
import functools

import jax
import jax.numpy as jnp
from jax import lax
from jax.experimental import pallas as pl
from jax.experimental.pallas import tpu as pltpu
from jax.experimental.pallas import tpu_sc as plsc

D = 128
NW = 32
CH = 128
NBUF = 4


def _gather_sc(idx2d, table, n):
    per_w = n // NW
    n_ch = per_w // CH
    m_iters = n_ch // NBUF

    mesh = plsc.VectorSubcoreMesh(core_axis_name="c", subcore_axis_name="s")

    @functools.partial(
        pl.kernel,
        mesh=mesh,
        out_type=jax.ShapeDtypeStruct((n, D), jnp.float32),
        scratch_types=[
            pltpu.VMEM((n_ch, CH), jnp.int32),
            pltpu.VMEM((NBUF, CH, D), jnp.float32),
            pltpu.SemaphoreType.DMA,
            pltpu.SemaphoreType.DMA,
            pltpu.SemaphoreType.DMA,
            pltpu.SemaphoreType.DMA,
            pltpu.SemaphoreType.DMA,
            pltpu.SemaphoreType.DMA,
            pltpu.SemaphoreType.DMA,
            pltpu.SemaphoreType.DMA,
        ],
    )
    def k(idx_hbm, table_hbm, out_hbm, idx_v, rows_v,
          g0, g1, g2, g3, s0, s1, s2, s3):
        gsem = (g0, g1, g2, g3)
        ssem = (s0, s1, s2, s3)
        wid = lax.axis_index("s") * 2 + lax.axis_index("c")
        base_ch = wid * n_ch

        pltpu.sync_copy(idx_hbm.at[pl.ds(base_ch, n_ch)], idx_v)

        def fire_gather(slot, i):
            return pltpu.async_copy(
                table_hbm.at[idx_v.at[i]], rows_v.at[slot], gsem[slot])

        def fire_store(slot, c):
            return pltpu.async_copy(
                rows_v.at[slot], out_hbm.at[pl.ds(c * CH, CH)], ssem[slot])

        def wait_store(slot):
            pltpu.make_async_copy(
                rows_v.at[slot], out_hbm.at[pl.ds(0, CH)], ssem[slot]).wait()

        def body(m, carry):
            i0 = m * NBUF
            c0 = base_ch + i0

            @pl.when(m > 0)
            def _():
                wait_store(0)
                wait_store(1)

            ga = fire_gather(0, i0)
            gb = fire_gather(1, i0 + 1)

            @pl.when(m > 0)
            def _():
                wait_store(2)
                wait_store(3)

            gc = fire_gather(2, i0 + 2)
            gd = fire_gather(3, i0 + 3)

            ga.wait()
            fire_store(0, c0)
            gb.wait()
            fire_store(1, c0 + 1)
            gc.wait()
            fire_store(2, c0 + 2)
            gd.wait()
            fire_store(3, c0 + 3)
            return carry

        lax.fori_loop(0, m_iters, body, 0)
        for slot in range(NBUF):
            wait_store(slot)

    return k(idx2d, table)


def kernel(x, infile):
    b, t = x.shape
    n = b * t
    idx2d = x.reshape(n // CH, CH).astype(jnp.int32)
    out = _gather_sc(idx2d, infile, n)
    return out.reshape(b, t, D)


# speedup vs baseline: 9.1262x; 1.0061x over previous
import functools

import jax
import jax.numpy as jnp
from jax import lax
from jax.experimental import pallas as pl
from jax.experimental.pallas import tpu as pltpu
from jax.experimental.pallas import tpu_sc as plsc

D = 128
NW = 32
CH = 128
NBUF = 5


def _gather_sc(idx2d, table, n):
    per_w = n // NW
    n_ch = per_w // CH
    m_iters = n_ch // NBUF

    mesh = plsc.VectorSubcoreMesh(core_axis_name="c", subcore_axis_name="s")

    @functools.partial(
        pl.kernel,
        mesh=mesh,
        out_type=jax.ShapeDtypeStruct((n, D), jnp.float32),
        scratch_types=(
            [pltpu.VMEM((n_ch, CH), jnp.int32),
             pltpu.VMEM((NBUF, CH, D), jnp.float32)]
            + [pltpu.SemaphoreType.DMA] * (2 * NBUF)
        ),
    )
    def k(idx_hbm, table_hbm, out_hbm, idx_v, rows_v, *sems):
        gsem = sems[:NBUF]
        ssem = sems[NBUF:]
        wid = lax.axis_index("s") * 2 + lax.axis_index("c")
        base_ch = wid * n_ch

        pltpu.sync_copy(idx_hbm.at[pl.ds(base_ch, n_ch)], idx_v)

        def fire_gather(slot, i):
            return pltpu.async_copy(
                table_hbm.at[idx_v.at[i]], rows_v.at[slot], gsem[slot])

        def fire_store(slot, c):
            return pltpu.async_copy(
                rows_v.at[slot], out_hbm.at[pl.ds(c * CH, CH)], ssem[slot])

        def wait_store(slot):
            pltpu.make_async_copy(
                rows_v.at[slot], out_hbm.at[pl.ds(0, CH)], ssem[slot]).wait()

        def body(m, carry):
            i0 = m * NBUF
            c0 = base_ch + i0

            descs = []
            for slot in range(NBUF):
                @pl.when(m > 0)
                def _(slot=slot):
                    wait_store(slot)

                descs.append(fire_gather(slot, i0 + slot))
            for slot in range(NBUF):
                descs[slot].wait()
                fire_store(slot, c0 + slot)
            return carry

        lax.fori_loop(0, m_iters, body, 0)
        for slot in range(NBUF):
            wait_store(slot)

    return k(idx2d, table)


def kernel(x, infile):
    b, t = x.shape
    n = b * t
    idx2d = x.reshape(n // CH, CH).astype(jnp.int32)
    out = _gather_sc(idx2d, infile, n)
    return out.reshape(b, t, D)
